# baseline (speedup 1.0000x reference)
import jax
import jax.numpy as jnp
from jax import lax
from jax.experimental import pallas as pl
from jax.experimental.pallas import tpu as pltpu
from jax.experimental.pallas import tpu_sc as plsc

H = 64
TW = 128
NC = 2
NS = 16
NW = NC * NS
CH = 128


ECH = 64


def _sc_edge_call(t, c, src, dst, n_pad, e_pad):
    npw = e_pad // (NW * ECH)
    assert npw * NW * ECH == e_pad and npw % 6 == 0 and npw >= 6
    rows_per_tile = n_pad // NS
    assert rows_per_tile * NS == n_pad and rows_per_tile % 8 == 0
    zch = ECH // 2
    zsizes = []
    off = 0
    while off < rows_per_tile:
        zsizes.append(min(zch, rows_per_tile - off))
        off += zch

    def body(t_hbm, c_hbm, src_hbm, dst_hbm, out_hbm,
             srcv0, dstv0, srcv1, dstv1, srcv2, dstv2,
             bufs0, bufd0, bufc0, bufs1, bufd1, bufc1,
             shared,
             si0, si1, si2, sa0, sb0, sc0, sa1, sb1, sc1, sd0, sd1):
        cid = lax.axis_index("c")
        sid = lax.axis_index("s")
        wid = sid * NC + cid
        zero16 = jnp.zeros((16,), jnp.float32)
        isets = [(srcv0, dstv0, si0), (srcv1, dstv1, si1), (srcv2, dstv2, si2)]
        dsets = [(bufs0, bufd0, bufc0, sa0, sb0, sc0, sd0),
                 (bufs1, bufd1, bufc1, sa1, sb1, sc1, sd1)]

        def zrow(i, _):
            for j in range(TW // 16):
                bufc0[i, pl.ds(j * 16, 16)] = zero16
            return 0
        lax.fori_loop(0, zch, zrow, 0)
        for k, zs in enumerate(zsizes):
            pltpu.sync_copy(bufc0.at[pl.ds(0, zs)],
                            shared.at[pl.ds(sid * rows_per_tile + k * zch, zs)])
        plsc.subcore_barrier()

        def issue_idx(j, ki):
            srcv, dstv, si = isets[ki]
            base = (wid * npw + j) * ECH
            pltpu.async_copy(src_hbm.at[pl.ds(base, ECH)], srcv, si)
            pltpu.async_copy(dst_hbm.at[pl.ds(base, ECH)], dstv, si)

        def wait_idx(ki):
            srcv, dstv, si = isets[ki]
            pltpu.make_async_copy(src_hbm.at[pl.ds(0, ECH)], srcv, si).wait()
            pltpu.make_async_copy(dst_hbm.at[pl.ds(0, ECH)], dstv, si).wait()

        def issue_gathers(j, kb, ki):
            srcv, dstv, _ = isets[ki]
            bufs, bufd, bufc, sa, sb, sc, _ = dsets[kb]
            pltpu.async_copy(t_hbm.at[srcv], bufs, sa)
            pltpu.async_copy(t_hbm.at[dstv], bufd, sb)
            base2 = (wid * npw + j) * (ECH // 2)
            pltpu.async_copy(c_hbm.at[pl.ds(base2, ECH // 2)], bufc, sc)

        def wait_gathers(kb, ki):
            srcv, dstv, _ = isets[ki]
            bufs, bufd, bufc, sa, sb, sc, _ = dsets[kb]
            pltpu.make_async_copy(t_hbm.at[srcv], bufs, sa).wait()
            pltpu.make_async_copy(t_hbm.at[dstv], bufd, sb).wait()
            pltpu.make_async_copy(c_hbm.at[pl.ds(0, ECH // 2)], bufc, sc).wait()

        def drain_scatter(kb, ki):
            _, dstv, _ = isets[ki]
            bufs, _, _, _, _, _, sd = dsets[kb]
            pltpu.make_async_copy(bufs, shared.at[dstv], sd).wait()

        def compute(kb, ki):
            _, dstv, _ = isets[ki]
            bufs, bufd, bufc, _, _, _, sd = dsets[kb]

            def row(i2, _):
                for half in range(2):
                    i = 2 * i2 + half
                    for j2 in range(H // 16):
                        sl = pl.ds(j2 * 16, 16)
                        v = (bufs[i, sl] + bufd[i, pl.ds(H + j2 * 16, 16)]
                             + bufc[i2, pl.ds(half * H + j2 * 16, 16)])
                        bufs[i, sl] = jnp.maximum(v, 0.0)
                return 0
            lax.fori_loop(0, ECH // 2, row, 0)
            pltpu.async_copy(bufs, shared.at[dstv], sd, add=True)

        issue_idx(0, 0)
        wait_idx(0)
        issue_gathers(0, 0, 0)
        issue_idx(1, 1)

        def outer(it, _):
            for p in range(6):
                j6 = 6 * it + p
                kb, ki = p % 2, p % 3
                ko = (p + 2) % 3
                wait_gathers(kb, ki)

                @pl.when(j6 >= 1)
                def _():
                    drain_scatter(1 - kb, ko)

                @pl.when(j6 + 2 < npw)
                def _():
                    issue_idx(j6 + 2, ko)

                @pl.when(j6 + 1 < npw)
                def _():
                    wait_idx((p + 1) % 3)
                    issue_gathers(j6 + 1, 1 - kb, (p + 1) % 3)
                compute(kb, ki)
            return 0
        lax.fori_loop(0, npw // 6, outer, 0)
        drain_scatter(1, 2)

        plsc.subcore_barrier()
        pltpu.sync_copy(shared.at[pl.ds(sid * rows_per_tile, rows_per_tile)],
                        out_hbm.at[cid, pl.ds(sid * rows_per_tile, rows_per_tile)])

    mesh = plsc.VectorSubcoreMesh(core_axis_name="c", subcore_axis_name="s",
                                  num_cores=NC, num_subcores=NS)
    idx_scratch = [pltpu.VMEM((ECH,), jnp.int32)] * 6
    data_scratch = [
        pltpu.VMEM((ECH, TW), jnp.float32),
        pltpu.VMEM((ECH, TW), jnp.float32),
        pltpu.VMEM((ECH // 2, TW), jnp.float32),
    ] * 2
    f = pl.kernel(
        body,
        out_type=jax.ShapeDtypeStruct((NC, n_pad, TW), jnp.float32),
        mesh=mesh,
        scratch_types=(
            idx_scratch + data_scratch
            + [pltpu.VMEM_SHARED((n_pad, TW), jnp.float32)]
            + [pltpu.SemaphoreType.DMA] * 11
        ),
    )
    return f(t, c, src, dst)


def _sc_degree_call(dst, n_pad, e_pad):
    npw = e_pad // (NW * CH)
    rows_per_tile = n_pad // NS
    zsizes = []
    off = 0
    while off < rows_per_tile:
        zsizes.append(min(CH, rows_per_tile - off))
        off += CH

    def body(dst_hbm, out_hbm, dstv, ones, shared):
        cid = lax.axis_index("c")
        sid = lax.axis_index("s")
        wid = sid * NC + cid
        zero16 = jnp.zeros((16,), jnp.float32)
        one16 = jnp.ones((16,), jnp.float32)

        def zrow(i, _):
            for j in range(TW // 16):
                ones[i, pl.ds(j * 16, 16)] = zero16
            return 0
        lax.fori_loop(0, CH, zrow, 0)
        for k, zs in enumerate(zsizes):
            pltpu.sync_copy(ones.at[pl.ds(0, zs)],
                            shared.at[pl.ds(sid * rows_per_tile + k * CH, zs)])

        def orow(i, _):
            for j in range(TW // 16):
                ones[i, pl.ds(j * 16, 16)] = one16
            return 0
        lax.fori_loop(0, CH, orow, 0)
        plsc.subcore_barrier()

        def chunk_body(j, _):
            base = (wid * npw + j) * CH
            pltpu.sync_copy(dst_hbm.at[pl.ds(base, CH)], dstv)
            pltpu.sync_copy(ones, shared.at[dstv], add=True)
            return 0
        lax.fori_loop(0, npw, chunk_body, 0)

        plsc.subcore_barrier()
        pltpu.sync_copy(shared.at[pl.ds(sid * rows_per_tile, rows_per_tile)],
                        out_hbm.at[cid, pl.ds(sid * rows_per_tile, rows_per_tile)])

    mesh = plsc.VectorSubcoreMesh(core_axis_name="c", subcore_axis_name="s",
                                  num_cores=NC, num_subcores=NS)
    f = pl.kernel(
        body,
        out_type=jax.ShapeDtypeStruct((NC, n_pad, TW), jnp.float32),
        mesh=mesh,
        scratch_types=[
            pltpu.VMEM((CH,), jnp.int32),
            pltpu.VMEM((CH, TW), jnp.float32),
            pltpu.VMEM_SHARED((n_pad, TW), jnp.float32),
        ],
    )
    return f(dst)


def _tc_pre_call(ent, We, be, Wm1sd, n, n_pad, blk):
    grid = n // blk
    d_in = ent.shape[1]

    def body(ent_ref, we_ref, be_ref, wsd_ref, x_ref, t_ref):
        x = jnp.dot(ent_ref[...], we_ref[...],
                    preferred_element_type=jnp.float32) + be_ref[...]
        x_ref[...] = x
        t_ref[...] = jnp.dot(x, wsd_ref[...], preferred_element_type=jnp.float32)

    return pl.pallas_call(
        body,
        grid=(grid,),
        in_specs=[
            pl.BlockSpec((blk, d_in), lambda i: (i, 0)),
            pl.BlockSpec((d_in, H), lambda i: (0, 0)),
            pl.BlockSpec((1, H), lambda i: (0, 0)),
            pl.BlockSpec((H, TW), lambda i: (0, 0)),
        ],
        out_specs=[
            pl.BlockSpec((blk, H), lambda i: (i, 0)),
            pl.BlockSpec((blk, TW), lambda i: (i, 0)),
        ],
        out_shape=[jax.ShapeDtypeStruct((n, H), jnp.float32),
                   jax.ShapeDtypeStruct((n_pad, TW), jnp.float32)],
    )(ent, We, be, Wm1sd)


def _tc_relc_call(rel2, Wc2, bc2, rows, blk):
    grid = rows // blk
    d2 = rel2.shape[1]

    def body(rel_ref, wc_ref, bc_ref, c_ref):
        c_ref[...] = jnp.dot(rel_ref[...], wc_ref[...],
                             preferred_element_type=jnp.float32) + bc_ref[...]

    return pl.pallas_call(
        body,
        grid=(grid,),
        in_specs=[
            pl.BlockSpec((blk, d2), lambda i: (i, 0)),
            pl.BlockSpec((d2, TW), lambda i: (0, 0)),
            pl.BlockSpec((1, TW), lambda i: (0, 0)),
        ],
        out_specs=pl.BlockSpec((blk, TW), lambda i: (i, 0)),
        out_shape=jax.ShapeDtypeStruct((rows, TW), jnp.float32),
    )(rel2, Wc2, bc2)


def _tc_degb_call(d0, d1, bsa, bu1, n, blk):
    grid = n // blk

    def body(d0_ref, d1_ref, bsa_ref, bu1_ref, out_ref):
        deg = (d0_ref[...] + d1_ref[...])[:, 0:1]
        out_ref[...] = jnp.dot(deg, bsa_ref[...],
                               preferred_element_type=jnp.float32) + bu1_ref[...]

    return pl.pallas_call(
        body,
        grid=(grid,),
        in_specs=[
            pl.BlockSpec((blk, TW), lambda i: (i, 0)),
            pl.BlockSpec((blk, TW), lambda i: (i, 0)),
            pl.BlockSpec((1, H), lambda i: (0, 0)),
            pl.BlockSpec((1, H), lambda i: (0, 0)),
        ],
        out_specs=pl.BlockSpec((blk, H), lambda i: (i, 0)),
        out_shape=jax.ShapeDtypeStruct((n, H), jnp.float32),
    )(d0, d1, bsa, bu1)


def _tc_update_call(x, s0, s1, degb, Wu1x, Wsa, Wu2, bu2, Wm1sd, n, n_pad, blk):
    grid = n // blk

    def body(x_ref, s0_ref, s1_ref, degb_ref, wux_ref, wsa_ref, wu2_ref,
             bu2_ref, wsd_ref, xn_ref, t_ref):
        x = x_ref[...]
        s = s0_ref[...] + s1_ref[...]
        pre = (jnp.dot(x, wux_ref[...], preferred_element_type=jnp.float32)
               + jnp.dot(s, wsa_ref[...], preferred_element_type=jnp.float32)
               + degb_ref[...])
        hu = jnp.maximum(pre, 0.0)
        xn = jnp.dot(hu, wu2_ref[...],
                     preferred_element_type=jnp.float32) + bu2_ref[...] + x
        xn_ref[...] = xn
        t_ref[...] = jnp.dot(xn, wsd_ref[...],
                             preferred_element_type=jnp.float32)

    full = lambda i: (0, 0)
    row = lambda i: (i, 0)
    return pl.pallas_call(
        body,
        grid=(grid,),
        in_specs=[
            pl.BlockSpec((blk, H), row),
            pl.BlockSpec((blk, TW), row),
            pl.BlockSpec((blk, TW), row),
            pl.BlockSpec((blk, H), row),
            pl.BlockSpec((H, H), full),
            pl.BlockSpec((TW, H), full),
            pl.BlockSpec((H, H), full),
            pl.BlockSpec((1, H), full),
            pl.BlockSpec((H, TW), full),
        ],
        out_specs=[
            pl.BlockSpec((blk, H), row),
            pl.BlockSpec((blk, TW), row),
        ],
        out_shape=[jax.ShapeDtypeStruct((n, H), jnp.float32),
                   jax.ShapeDtypeStruct((n_pad, TW), jnp.float32)],
    )(x, s0, s1, degb, Wu1x, Wsa, Wu2, bu2, Wm1sd)


def kernel(entity_features, relation_features, edge_index, We, be, Wr, br,
           Wm1, bm1, Wm2, bm2, Wu1, bu1, Wu2, bu2):
    n = entity_features.shape[0]
    e = relation_features.shape[0]

    Wm1s, Wm1r, Wm1d = Wm1[:H], Wm1[H:2 * H], Wm1[2 * H:]
    Wm1sd = jnp.concatenate([Wm1s, Wm1d], axis=1)
    Wc = Wr @ Wm1r
    bc = br @ Wm1r + bm1
    d_rel = Wr.shape[0]
    zpad = jnp.zeros((d_rel, H), jnp.float32)
    Wc2 = jnp.concatenate([jnp.concatenate([Wc, zpad], axis=1),
                           jnp.concatenate([zpad, Wc], axis=1)], axis=0)
    bc2 = jnp.concatenate([bc, bc]).reshape(1, TW)
    Wu1x, Wu1a = Wu1[:H], Wu1[H:]
    Wsa = jnp.concatenate([Wm2 @ Wu1a, jnp.zeros((TW - H, H), jnp.float32)],
                          axis=0)
    bsa = (bm2 @ Wu1a).reshape(1, H)
    be2 = be.reshape(1, H)
    bu1r = bu1.reshape(1, H)
    bu2r = bu2.reshape(1, H)

    n_pad = ((n + 1 + 127) // 128) * 128
    npw = ((e + NW * ECH - 1) // (NW * ECH) + 5) // 6 * 6
    e_pad = NW * ECH * npw
    assert e_pad % (NW * CH) == 0
    src = jnp.concatenate([edge_index[0],
                           jnp.full((e_pad - e,), n, jnp.int32)])
    dst = jnp.concatenate([edge_index[1],
                           jnp.full((e_pad - e,), n, jnp.int32)])
    rel_p = jnp.concatenate([relation_features,
                             jnp.zeros((e_pad - e, relation_features.shape[1]),
                                       jnp.float32)])
    rel2 = rel_p.reshape(e_pad // 2, 2 * d_rel)

    x, t = _tc_pre_call(entity_features, We, be2, Wm1sd, n, n_pad, blk=1000)
    c = _tc_relc_call(rel2, Wc2, bc2, e_pad // 2, blk=2048)
    dpart = _sc_degree_call(dst, n_pad, e_pad)
    degb = _tc_degb_call(dpart[0], dpart[1], bsa, bu1r, n, blk=1000)

    for _ in range(3):
        s = _sc_edge_call(t, c, src, dst, n_pad, e_pad)
        x, t = _tc_update_call(x, s[0], s[1], degb, Wu1x, Wsa, Wu2, bu2r,
                               Wm1sd, n, n_pad, blk=1000)
    return x


# speedup vs baseline: 1.6114x; 1.6114x over previous
import jax
import jax.numpy as jnp
from jax import lax
from jax.experimental import pallas as pl
from jax.experimental.pallas import tpu as pltpu
from jax.experimental.pallas import tpu_sc as plsc

H = 64
TW = 128
NC = 2
NS = 16
NW = NC * NS
CH = 128


ECH = 64


def _sc_edge_call(t, c, src, dst, n_pad, e_pad):
    npw = e_pad // (NW * ECH)
    assert npw * NW * ECH == e_pad and npw % 2 == 0
    rows_per_tile = n_pad // NS
    assert rows_per_tile * NS == n_pad and rows_per_tile % 8 == 0
    zch = ECH // 2
    zsizes = []
    off = 0
    while off < rows_per_tile:
        zsizes.append(min(zch, rows_per_tile - off))
        off += zch

    def body(t_hbm, c_hbm, src_hbm, dst_hbm, out_hbm,
             srcv0, dstv0, bufs0, bufd0, bufc0,
             srcv1, dstv1, bufs1, bufd1, bufc1,
             shared, sa0, sb0, sc0, si0, sa1, sb1, sc1, si1):
        cid = lax.axis_index("c")
        sid = lax.axis_index("s")
        wid = sid * NC + cid
        zero16 = jnp.zeros((16,), jnp.float32)
        sets = [
            (srcv0, dstv0, bufs0, bufd0, bufc0, sa0, sb0, sc0, si0),
            (srcv1, dstv1, bufs1, bufd1, bufc1, sa1, sb1, sc1, si1),
        ]

        def zrow(i, _):
            for j in range(TW // 16):
                bufc0[i, pl.ds(j * 16, 16)] = zero16
            return 0
        lax.fori_loop(0, zch, zrow, 0)
        for k, zs in enumerate(zsizes):
            pltpu.sync_copy(bufc0.at[pl.ds(0, zs)],
                            shared.at[pl.ds(sid * rows_per_tile + k * zch, zs)])
        plsc.subcore_barrier()

        def start(j, s):
            srcv, dstv, bufs, bufd, bufc, sa, sb, sc, si = s
            base = (wid * npw + j) * ECH
            pltpu.async_copy(src_hbm.at[pl.ds(base, ECH)], srcv, si)
            pltpu.async_copy(dst_hbm.at[pl.ds(base, ECH)], dstv, si)
            pltpu.make_async_copy(src_hbm.at[pl.ds(0, ECH)], srcv, si).wait()
            pltpu.make_async_copy(dst_hbm.at[pl.ds(0, ECH)], dstv, si).wait()
            pltpu.async_copy(t_hbm.at[srcv], bufs, sa)
            pltpu.async_copy(t_hbm.at[dstv], bufd, sb)
            base2 = (wid * npw + j) * (ECH // 2)
            pltpu.async_copy(c_hbm.at[pl.ds(base2, ECH // 2)], bufc, sc)

        def waitg(s):
            srcv, dstv, bufs, bufd, bufc, sa, sb, sc, si = s
            pltpu.make_async_copy(t_hbm.at[srcv], bufs, sa).wait()
            pltpu.make_async_copy(t_hbm.at[dstv], bufd, sb).wait()
            pltpu.make_async_copy(c_hbm.at[pl.ds(0, ECH // 2)], bufc, sc).wait()

        def compute_scatter(s):
            srcv, dstv, bufs, bufd, bufc, sa, sb, sc, si = s

            def row(i2, _):
                for half in range(2):
                    i = 2 * i2 + half
                    for j2 in range(H // 16):
                        sl = pl.ds(j2 * 16, 16)
                        v = (bufs[i, sl] + bufd[i, pl.ds(H + j2 * 16, 16)]
                             + bufc[i2, pl.ds(half * H + j2 * 16, 16)])
                        bufs[i, sl] = jnp.maximum(v, 0.0)
                return 0
            lax.fori_loop(0, ECH // 2, row, 0)
            pltpu.sync_copy(bufs, shared.at[dstv], add=True)

        start(0, sets[0])

        def outer(it, _):
            jj = 2 * it
            for b in range(2):
                j = jj + b
                waitg(sets[b])

                @pl.when(j + 1 < npw)
                def _():
                    start(j + 1, sets[1 - b])
                compute_scatter(sets[b])
            return 0
        lax.fori_loop(0, npw // 2, outer, 0)

        plsc.subcore_barrier()
        pltpu.sync_copy(shared.at[pl.ds(sid * rows_per_tile, rows_per_tile)],
                        out_hbm.at[cid, pl.ds(sid * rows_per_tile, rows_per_tile)])

    mesh = plsc.VectorSubcoreMesh(core_axis_name="c", subcore_axis_name="s",
                                  num_cores=NC, num_subcores=NS)
    chunk_scratch = [
        pltpu.VMEM((ECH,), jnp.int32),
        pltpu.VMEM((ECH,), jnp.int32),
        pltpu.VMEM((ECH, TW), jnp.float32),
        pltpu.VMEM((ECH, TW), jnp.float32),
        pltpu.VMEM((ECH // 2, TW), jnp.float32),
    ]
    f = pl.kernel(
        body,
        out_type=jax.ShapeDtypeStruct((NC, n_pad, TW), jnp.float32),
        mesh=mesh,
        scratch_types=(
            chunk_scratch + chunk_scratch
            + [pltpu.VMEM_SHARED((n_pad, TW), jnp.float32)]
            + [pltpu.SemaphoreType.DMA] * 8
        ),
    )
    return f(t, c, src, dst)


def _sc_degree_call(dst, n_pad, e_pad):
    npw = e_pad // (NW * CH)
    rows_per_tile = n_pad // NS
    zsizes = []
    off = 0
    while off < rows_per_tile:
        zsizes.append(min(CH, rows_per_tile - off))
        off += CH

    def body(dst_hbm, out_hbm, dstv, ones, shared):
        cid = lax.axis_index("c")
        sid = lax.axis_index("s")
        wid = sid * NC + cid
        zero16 = jnp.zeros((16,), jnp.float32)
        one16 = jnp.ones((16,), jnp.float32)

        def zrow(i, _):
            for j in range(TW // 16):
                ones[i, pl.ds(j * 16, 16)] = zero16
            return 0
        lax.fori_loop(0, CH, zrow, 0)
        for k, zs in enumerate(zsizes):
            pltpu.sync_copy(ones.at[pl.ds(0, zs)],
                            shared.at[pl.ds(sid * rows_per_tile + k * CH, zs)])

        def orow(i, _):
            for j in range(TW // 16):
                ones[i, pl.ds(j * 16, 16)] = one16
            return 0
        lax.fori_loop(0, CH, orow, 0)
        plsc.subcore_barrier()

        def chunk_body(j, _):
            base = (wid * npw + j) * CH
            pltpu.sync_copy(dst_hbm.at[pl.ds(base, CH)], dstv)
            pltpu.sync_copy(ones, shared.at[dstv], add=True)
            return 0
        lax.fori_loop(0, npw, chunk_body, 0)

        plsc.subcore_barrier()
        pltpu.sync_copy(shared.at[pl.ds(sid * rows_per_tile, rows_per_tile)],
                        out_hbm.at[cid, pl.ds(sid * rows_per_tile, rows_per_tile)])

    mesh = plsc.VectorSubcoreMesh(core_axis_name="c", subcore_axis_name="s",
                                  num_cores=NC, num_subcores=NS)
    f = pl.kernel(
        body,
        out_type=jax.ShapeDtypeStruct((NC, n_pad, TW), jnp.float32),
        mesh=mesh,
        scratch_types=[
            pltpu.VMEM((CH,), jnp.int32),
            pltpu.VMEM((CH, TW), jnp.float32),
            pltpu.VMEM_SHARED((n_pad, TW), jnp.float32),
        ],
    )
    return f(dst)


def _tc_pre_call(ent, We, be, Wm1sd, n, n_pad, blk):
    grid = n // blk
    d_in = ent.shape[1]

    def body(ent_ref, we_ref, be_ref, wsd_ref, x_ref, t_ref):
        x = jnp.dot(ent_ref[...], we_ref[...],
                    preferred_element_type=jnp.float32) + be_ref[...]
        x_ref[...] = x
        t_ref[...] = jnp.dot(x, wsd_ref[...], preferred_element_type=jnp.float32)

    return pl.pallas_call(
        body,
        grid=(grid,),
        in_specs=[
            pl.BlockSpec((blk, d_in), lambda i: (i, 0)),
            pl.BlockSpec((d_in, H), lambda i: (0, 0)),
            pl.BlockSpec((1, H), lambda i: (0, 0)),
            pl.BlockSpec((H, TW), lambda i: (0, 0)),
        ],
        out_specs=[
            pl.BlockSpec((blk, H), lambda i: (i, 0)),
            pl.BlockSpec((blk, TW), lambda i: (i, 0)),
        ],
        out_shape=[jax.ShapeDtypeStruct((n, H), jnp.float32),
                   jax.ShapeDtypeStruct((n_pad, TW), jnp.float32)],
    )(ent, We, be, Wm1sd)


def _tc_relc_call(rel2, Wc2, bc2, rows, blk):
    grid = rows // blk
    d2 = rel2.shape[1]

    def body(rel_ref, wc_ref, bc_ref, c_ref):
        c_ref[...] = jnp.dot(rel_ref[...], wc_ref[...],
                             preferred_element_type=jnp.float32) + bc_ref[...]

    return pl.pallas_call(
        body,
        grid=(grid,),
        in_specs=[
            pl.BlockSpec((blk, d2), lambda i: (i, 0)),
            pl.BlockSpec((d2, TW), lambda i: (0, 0)),
            pl.BlockSpec((1, TW), lambda i: (0, 0)),
        ],
        out_specs=pl.BlockSpec((blk, TW), lambda i: (i, 0)),
        out_shape=jax.ShapeDtypeStruct((rows, TW), jnp.float32),
    )(rel2, Wc2, bc2)


def _tc_degb_call(d0, d1, bsa, bu1, n, blk):
    grid = n // blk

    def body(d0_ref, d1_ref, bsa_ref, bu1_ref, out_ref):
        deg = (d0_ref[...] + d1_ref[...])[:, 0:1]
        out_ref[...] = jnp.dot(deg, bsa_ref[...],
                               preferred_element_type=jnp.float32) + bu1_ref[...]

    return pl.pallas_call(
        body,
        grid=(grid,),
        in_specs=[
            pl.BlockSpec((blk, TW), lambda i: (i, 0)),
            pl.BlockSpec((blk, TW), lambda i: (i, 0)),
            pl.BlockSpec((1, H), lambda i: (0, 0)),
            pl.BlockSpec((1, H), lambda i: (0, 0)),
        ],
        out_specs=pl.BlockSpec((blk, H), lambda i: (i, 0)),
        out_shape=jax.ShapeDtypeStruct((n, H), jnp.float32),
    )(d0, d1, bsa, bu1)


def _tc_update_call(x, s0, s1, degb, Wu1x, Wsa, Wu2, bu2, Wm1sd, n, n_pad, blk):
    grid = n // blk

    def body(x_ref, s0_ref, s1_ref, degb_ref, wux_ref, wsa_ref, wu2_ref,
             bu2_ref, wsd_ref, xn_ref, t_ref):
        x = x_ref[...]
        s = s0_ref[...] + s1_ref[...]
        pre = (jnp.dot(x, wux_ref[...], preferred_element_type=jnp.float32)
               + jnp.dot(s, wsa_ref[...], preferred_element_type=jnp.float32)
               + degb_ref[...])
        hu = jnp.maximum(pre, 0.0)
        xn = jnp.dot(hu, wu2_ref[...],
                     preferred_element_type=jnp.float32) + bu2_ref[...] + x
        xn_ref[...] = xn
        t_ref[...] = jnp.dot(xn, wsd_ref[...],
                             preferred_element_type=jnp.float32)

    full = lambda i: (0, 0)
    row = lambda i: (i, 0)
    return pl.pallas_call(
        body,
        grid=(grid,),
        in_specs=[
            pl.BlockSpec((blk, H), row),
            pl.BlockSpec((blk, TW), row),
            pl.BlockSpec((blk, TW), row),
            pl.BlockSpec((blk, H), row),
            pl.BlockSpec((H, H), full),
            pl.BlockSpec((TW, H), full),
            pl.BlockSpec((H, H), full),
            pl.BlockSpec((1, H), full),
            pl.BlockSpec((H, TW), full),
        ],
        out_specs=[
            pl.BlockSpec((blk, H), row),
            pl.BlockSpec((blk, TW), row),
        ],
        out_shape=[jax.ShapeDtypeStruct((n, H), jnp.float32),
                   jax.ShapeDtypeStruct((n_pad, TW), jnp.float32)],
    )(x, s0, s1, degb, Wu1x, Wsa, Wu2, bu2, Wm1sd)


def kernel(entity_features, relation_features, edge_index, We, be, Wr, br,
           Wm1, bm1, Wm2, bm2, Wu1, bu1, Wu2, bu2):
    n = entity_features.shape[0]
    e = relation_features.shape[0]

    Wm1s, Wm1r, Wm1d = Wm1[:H], Wm1[H:2 * H], Wm1[2 * H:]
    Wm1sd = jnp.concatenate([Wm1s, Wm1d], axis=1)
    Wc = Wr @ Wm1r
    bc = br @ Wm1r + bm1
    d_rel = Wr.shape[0]
    zpad = jnp.zeros((d_rel, H), jnp.float32)
    Wc2 = jnp.concatenate([jnp.concatenate([Wc, zpad], axis=1),
                           jnp.concatenate([zpad, Wc], axis=1)], axis=0)
    bc2 = jnp.concatenate([bc, bc]).reshape(1, TW)
    Wu1x, Wu1a = Wu1[:H], Wu1[H:]
    Wsa = jnp.concatenate([Wm2 @ Wu1a, jnp.zeros((TW - H, H), jnp.float32)],
                          axis=0)
    bsa = (bm2 @ Wu1a).reshape(1, H)
    be2 = be.reshape(1, H)
    bu1r = bu1.reshape(1, H)
    bu2r = bu2.reshape(1, H)

    n_pad = ((n + 1 + 127) // 128) * 128
    e_pad = ((e + NW * CH - 1) // (NW * CH)) * (NW * CH)
    src = jnp.concatenate([edge_index[0],
                           jnp.full((e_pad - e,), n, jnp.int32)])
    dst = jnp.concatenate([edge_index[1],
                           jnp.full((e_pad - e,), n, jnp.int32)])
    rel_p = jnp.concatenate([relation_features,
                             jnp.zeros((e_pad - e, relation_features.shape[1]),
                                       jnp.float32)])
    rel2 = rel_p.reshape(e_pad // 2, 2 * d_rel)

    x, t = _tc_pre_call(entity_features, We, be2, Wm1sd, n, n_pad, blk=1000)
    c = _tc_relc_call(rel2, Wc2, bc2, e_pad // 2, blk=2048)
    dpart = _sc_degree_call(dst, n_pad, e_pad)
    degb = _tc_degb_call(dpart[0], dpart[1], bsa, bu1r, n, blk=1000)

    for _ in range(3):
        s = _sc_edge_call(t, c, src, dst, n_pad, e_pad)
        x, t = _tc_update_call(x, s[0], s[1], degb, Wu1x, Wsa, Wu2, bu2r,
                               Wm1sd, n, n_pad, blk=1000)
    return x


# speedup vs baseline: 1.6129x; 1.0009x over previous
import jax
import jax.numpy as jnp
from jax import lax
from jax.experimental import pallas as pl
from jax.experimental.pallas import tpu as pltpu
from jax.experimental.pallas import tpu_sc as plsc

H = 64
TW = 128
NC = 2
NS = 16
NW = NC * NS
CH = 128


ECH = 64


def _sc_edge_call(t, c, src, dst, n_pad, e_pad):
    npw = e_pad // (NW * ECH)
    assert npw * NW * ECH == e_pad and npw % 2 == 0
    rows_per_tile = n_pad // NS
    assert rows_per_tile * NS == n_pad and rows_per_tile % 8 == 0
    zch = ECH // 2
    zsizes = []
    off = 0
    while off < rows_per_tile:
        zsizes.append(min(zch, rows_per_tile - off))
        off += zch

    def body(t_hbm, c_hbm, src_hbm, dst_hbm, out_hbm,
             srcv0, dstv0, bufs0, bufd0, bufc0,
             srcv1, dstv1, bufs1, bufd1, bufc1,
             shared, sa0, sb0, sc0, si0, sd0, sa1, sb1, sc1, si1, sd1):
        cid = lax.axis_index("c")
        sid = lax.axis_index("s")
        wid = sid * NC + cid
        zero16 = jnp.zeros((16,), jnp.float32)
        sets = [
            (srcv0, dstv0, bufs0, bufd0, bufc0, sa0, sb0, sc0, si0, sd0),
            (srcv1, dstv1, bufs1, bufd1, bufc1, sa1, sb1, sc1, si1, sd1),
        ]

        def zrow(i, _):
            for j in range(TW // 16):
                bufc0[i, pl.ds(j * 16, 16)] = zero16
            return 0
        lax.fori_loop(0, zch, zrow, 0)
        for k, zs in enumerate(zsizes):
            pltpu.sync_copy(bufc0.at[pl.ds(0, zs)],
                            shared.at[pl.ds(sid * rows_per_tile + k * zch, zs)])
        plsc.subcore_barrier()

        def start(j, s):
            srcv, dstv, bufs, bufd, bufc, sa, sb, sc, si, sd = s

            @pl.when(j >= 2)
            def _():
                pltpu.make_async_copy(bufs, shared.at[dstv], sd).wait()
            base = (wid * npw + j) * ECH
            pltpu.async_copy(src_hbm.at[pl.ds(base, ECH)], srcv, si)
            pltpu.async_copy(dst_hbm.at[pl.ds(base, ECH)], dstv, si)
            pltpu.make_async_copy(src_hbm.at[pl.ds(0, ECH)], srcv, si).wait()
            pltpu.make_async_copy(dst_hbm.at[pl.ds(0, ECH)], dstv, si).wait()
            pltpu.async_copy(t_hbm.at[srcv], bufs, sa)
            pltpu.async_copy(t_hbm.at[dstv], bufd, sb)
            base2 = (wid * npw + j) * (ECH // 2)
            pltpu.async_copy(c_hbm.at[pl.ds(base2, ECH // 2)], bufc, sc)

        def waitg(s):
            srcv, dstv, bufs, bufd, bufc, sa, sb, sc, si, sd = s
            pltpu.make_async_copy(t_hbm.at[srcv], bufs, sa).wait()
            pltpu.make_async_copy(t_hbm.at[dstv], bufd, sb).wait()
            pltpu.make_async_copy(c_hbm.at[pl.ds(0, ECH // 2)], bufc, sc).wait()

        def compute_scatter(s):
            srcv, dstv, bufs, bufd, bufc, sa, sb, sc, si, sd = s

            def row(i2, _):
                for half in range(2):
                    i = 2 * i2 + half
                    for j2 in range(H // 16):
                        sl = pl.ds(j2 * 16, 16)
                        v = (bufs[i, sl] + bufd[i, pl.ds(H + j2 * 16, 16)]
                             + bufc[i2, pl.ds(half * H + j2 * 16, 16)])
                        bufs[i, sl] = jnp.maximum(v, 0.0)
                return 0
            lax.fori_loop(0, ECH // 2, row, 0)
            pltpu.async_copy(bufs, shared.at[dstv], sd, add=True)

        start(0, sets[0])

        def outer(it, _):
            jj = 2 * it
            for b in range(2):
                j = jj + b
                waitg(sets[b])

                @pl.when(j + 1 < npw)
                def _():
                    start(j + 1, sets[1 - b])
                compute_scatter(sets[b])
            return 0
        lax.fori_loop(0, npw // 2, outer, 0)
        for s in sets:
            srcv, dstv, bufs, bufd, bufc, sa, sb, sc, si, sd = s
            pltpu.make_async_copy(bufs, shared.at[dstv], sd).wait()

        plsc.subcore_barrier()
        pltpu.sync_copy(shared.at[pl.ds(sid * rows_per_tile, rows_per_tile)],
                        out_hbm.at[cid, pl.ds(sid * rows_per_tile, rows_per_tile)])

    mesh = plsc.VectorSubcoreMesh(core_axis_name="c", subcore_axis_name="s",
                                  num_cores=NC, num_subcores=NS)
    chunk_scratch = [
        pltpu.VMEM((ECH,), jnp.int32),
        pltpu.VMEM((ECH,), jnp.int32),
        pltpu.VMEM((ECH, TW), jnp.float32),
        pltpu.VMEM((ECH, TW), jnp.float32),
        pltpu.VMEM((ECH // 2, TW), jnp.float32),
    ]
    f = pl.kernel(
        body,
        out_type=jax.ShapeDtypeStruct((NC, n_pad, TW), jnp.float32),
        mesh=mesh,
        scratch_types=(
            chunk_scratch + chunk_scratch
            + [pltpu.VMEM_SHARED((n_pad, TW), jnp.float32)]
            + [pltpu.SemaphoreType.DMA] * 10
        ),
    )
    return f(t, c, src, dst)


def _sc_degree_call(dst, n_pad, e_pad):
    npw = e_pad // (NW * CH)
    rows_per_tile = n_pad // NS
    zsizes = []
    off = 0
    while off < rows_per_tile:
        zsizes.append(min(CH, rows_per_tile - off))
        off += CH

    def body(dst_hbm, out_hbm, dstv, ones, shared):
        cid = lax.axis_index("c")
        sid = lax.axis_index("s")
        wid = sid * NC + cid
        zero16 = jnp.zeros((16,), jnp.float32)
        one16 = jnp.ones((16,), jnp.float32)

        def zrow(i, _):
            for j in range(TW // 16):
                ones[i, pl.ds(j * 16, 16)] = zero16
            return 0
        lax.fori_loop(0, CH, zrow, 0)
        for k, zs in enumerate(zsizes):
            pltpu.sync_copy(ones.at[pl.ds(0, zs)],
                            shared.at[pl.ds(sid * rows_per_tile + k * CH, zs)])

        def orow(i, _):
            for j in range(TW // 16):
                ones[i, pl.ds(j * 16, 16)] = one16
            return 0
        lax.fori_loop(0, CH, orow, 0)
        plsc.subcore_barrier()

        def chunk_body(j, _):
            base = (wid * npw + j) * CH
            pltpu.sync_copy(dst_hbm.at[pl.ds(base, CH)], dstv)
            pltpu.sync_copy(ones, shared.at[dstv], add=True)
            return 0
        lax.fori_loop(0, npw, chunk_body, 0)

        plsc.subcore_barrier()
        pltpu.sync_copy(shared.at[pl.ds(sid * rows_per_tile, rows_per_tile)],
                        out_hbm.at[cid, pl.ds(sid * rows_per_tile, rows_per_tile)])

    mesh = plsc.VectorSubcoreMesh(core_axis_name="c", subcore_axis_name="s",
                                  num_cores=NC, num_subcores=NS)
    f = pl.kernel(
        body,
        out_type=jax.ShapeDtypeStruct((NC, n_pad, TW), jnp.float32),
        mesh=mesh,
        scratch_types=[
            pltpu.VMEM((CH,), jnp.int32),
            pltpu.VMEM((CH, TW), jnp.float32),
            pltpu.VMEM_SHARED((n_pad, TW), jnp.float32),
        ],
    )
    return f(dst)


def _tc_pre_call(ent, We, be, Wm1sd, n, n_pad, blk):
    grid = n // blk
    d_in = ent.shape[1]

    def body(ent_ref, we_ref, be_ref, wsd_ref, x_ref, t_ref):
        x = jnp.dot(ent_ref[...], we_ref[...],
                    preferred_element_type=jnp.float32) + be_ref[...]
        x_ref[...] = x
        t_ref[...] = jnp.dot(x, wsd_ref[...], preferred_element_type=jnp.float32)

    return pl.pallas_call(
        body,
        grid=(grid,),
        in_specs=[
            pl.BlockSpec((blk, d_in), lambda i: (i, 0)),
            pl.BlockSpec((d_in, H), lambda i: (0, 0)),
            pl.BlockSpec((1, H), lambda i: (0, 0)),
            pl.BlockSpec((H, TW), lambda i: (0, 0)),
        ],
        out_specs=[
            pl.BlockSpec((blk, H), lambda i: (i, 0)),
            pl.BlockSpec((blk, TW), lambda i: (i, 0)),
        ],
        out_shape=[jax.ShapeDtypeStruct((n, H), jnp.float32),
                   jax.ShapeDtypeStruct((n_pad, TW), jnp.float32)],
    )(ent, We, be, Wm1sd)


def _tc_relc_call(rel2, Wc2, bc2, rows, blk):
    grid = rows // blk
    d2 = rel2.shape[1]

    def body(rel_ref, wc_ref, bc_ref, c_ref):
        c_ref[...] = jnp.dot(rel_ref[...], wc_ref[...],
                             preferred_element_type=jnp.float32) + bc_ref[...]

    return pl.pallas_call(
        body,
        grid=(grid,),
        in_specs=[
            pl.BlockSpec((blk, d2), lambda i: (i, 0)),
            pl.BlockSpec((d2, TW), lambda i: (0, 0)),
            pl.BlockSpec((1, TW), lambda i: (0, 0)),
        ],
        out_specs=pl.BlockSpec((blk, TW), lambda i: (i, 0)),
        out_shape=jax.ShapeDtypeStruct((rows, TW), jnp.float32),
    )(rel2, Wc2, bc2)


def _tc_degb_call(d0, d1, bsa, bu1, n, blk):
    grid = n // blk

    def body(d0_ref, d1_ref, bsa_ref, bu1_ref, out_ref):
        deg = (d0_ref[...] + d1_ref[...])[:, 0:1]
        out_ref[...] = jnp.dot(deg, bsa_ref[...],
                               preferred_element_type=jnp.float32) + bu1_ref[...]

    return pl.pallas_call(
        body,
        grid=(grid,),
        in_specs=[
            pl.BlockSpec((blk, TW), lambda i: (i, 0)),
            pl.BlockSpec((blk, TW), lambda i: (i, 0)),
            pl.BlockSpec((1, H), lambda i: (0, 0)),
            pl.BlockSpec((1, H), lambda i: (0, 0)),
        ],
        out_specs=pl.BlockSpec((blk, H), lambda i: (i, 0)),
        out_shape=jax.ShapeDtypeStruct((n, H), jnp.float32),
    )(d0, d1, bsa, bu1)


def _tc_update_call(x, s0, s1, degb, Wu1x, Wsa, Wu2, bu2, Wm1sd, n, n_pad, blk):
    grid = n // blk

    def body(x_ref, s0_ref, s1_ref, degb_ref, wux_ref, wsa_ref, wu2_ref,
             bu2_ref, wsd_ref, xn_ref, t_ref):
        x = x_ref[...]
        s = s0_ref[...] + s1_ref[...]
        pre = (jnp.dot(x, wux_ref[...], preferred_element_type=jnp.float32)
               + jnp.dot(s, wsa_ref[...], preferred_element_type=jnp.float32)
               + degb_ref[...])
        hu = jnp.maximum(pre, 0.0)
        xn = jnp.dot(hu, wu2_ref[...],
                     preferred_element_type=jnp.float32) + bu2_ref[...] + x
        xn_ref[...] = xn
        t_ref[...] = jnp.dot(xn, wsd_ref[...],
                             preferred_element_type=jnp.float32)

    full = lambda i: (0, 0)
    row = lambda i: (i, 0)
    return pl.pallas_call(
        body,
        grid=(grid,),
        in_specs=[
            pl.BlockSpec((blk, H), row),
            pl.BlockSpec((blk, TW), row),
            pl.BlockSpec((blk, TW), row),
            pl.BlockSpec((blk, H), row),
            pl.BlockSpec((H, H), full),
            pl.BlockSpec((TW, H), full),
            pl.BlockSpec((H, H), full),
            pl.BlockSpec((1, H), full),
            pl.BlockSpec((H, TW), full),
        ],
        out_specs=[
            pl.BlockSpec((blk, H), row),
            pl.BlockSpec((blk, TW), row),
        ],
        out_shape=[jax.ShapeDtypeStruct((n, H), jnp.float32),
                   jax.ShapeDtypeStruct((n_pad, TW), jnp.float32)],
    )(x, s0, s1, degb, Wu1x, Wsa, Wu2, bu2, Wm1sd)


def kernel(entity_features, relation_features, edge_index, We, be, Wr, br,
           Wm1, bm1, Wm2, bm2, Wu1, bu1, Wu2, bu2):
    n = entity_features.shape[0]
    e = relation_features.shape[0]

    Wm1s, Wm1r, Wm1d = Wm1[:H], Wm1[H:2 * H], Wm1[2 * H:]
    Wm1sd = jnp.concatenate([Wm1s, Wm1d], axis=1)
    Wc = Wr @ Wm1r
    bc = br @ Wm1r + bm1
    d_rel = Wr.shape[0]
    zpad = jnp.zeros((d_rel, H), jnp.float32)
    Wc2 = jnp.concatenate([jnp.concatenate([Wc, zpad], axis=1),
                           jnp.concatenate([zpad, Wc], axis=1)], axis=0)
    bc2 = jnp.concatenate([bc, bc]).reshape(1, TW)
    Wu1x, Wu1a = Wu1[:H], Wu1[H:]
    Wsa = jnp.concatenate([Wm2 @ Wu1a, jnp.zeros((TW - H, H), jnp.float32)],
                          axis=0)
    bsa = (bm2 @ Wu1a).reshape(1, H)
    be2 = be.reshape(1, H)
    bu1r = bu1.reshape(1, H)
    bu2r = bu2.reshape(1, H)

    n_pad = ((n + 1 + 127) // 128) * 128
    e_pad = ((e + NW * CH - 1) // (NW * CH)) * (NW * CH)
    src = jnp.concatenate([edge_index[0],
                           jnp.full((e_pad - e,), n, jnp.int32)])
    dst = jnp.concatenate([edge_index[1],
                           jnp.full((e_pad - e,), n, jnp.int32)])
    rel_p = jnp.concatenate([relation_features,
                             jnp.zeros((e_pad - e, relation_features.shape[1]),
                                       jnp.float32)])
    rel2 = rel_p.reshape(e_pad // 2, 2 * d_rel)

    x, t = _tc_pre_call(entity_features, We, be2, Wm1sd, n, n_pad, blk=1000)
    c = _tc_relc_call(rel2, Wc2, bc2, e_pad // 2, blk=2048)
    dpart = _sc_degree_call(dst, n_pad, e_pad)
    degb = _tc_degb_call(dpart[0], dpart[1], bsa, bu1r, n, blk=1000)

    for _ in range(3):
        s = _sc_edge_call(t, c, src, dst, n_pad, e_pad)
        x, t = _tc_update_call(x, s[0], s[1], degb, Wu1x, Wsa, Wu2, bu2r,
                               Wm1sd, n, n_pad, blk=1000)
    return x
